# baseline (speedup 1.0000x reference)
import functools

import jax
import jax.numpy as jnp
from jax import lax
from jax.experimental import pallas as pl
from jax.experimental.pallas import tpu as pltpu
from jax.experimental.pallas import tpu_sc as plsc

CODEBOOK = 8192
DIM = 32
ROWS = 16384
BLK = 512
KC = 1024
NBLK = ROWS // BLK
NKC = CODEBOOK // KC


def _argmin_body(z_ref, zsq_ref, wt_ref, wsq_ref, idx_ref, minv_ref):
    z = z_ref[...]
    zsq = zsq_ref[...]

    def step(j, carry):
        rmin, ridx = carry
        wt = wt_ref[j]
        wsq = wsq_ref[j]
        dist = (zsq - jnp.dot(z, wt, preferred_element_type=jnp.float32)) + wsq
        cm = jnp.min(dist, axis=1, keepdims=True)
        io = lax.broadcasted_iota(jnp.int32, (BLK, KC), 1)
        ci = jnp.min(jnp.where(dist == cm, io, CODEBOOK),
                     axis=1, keepdims=True) + j * KC
        upd = cm < rmin
        return jnp.where(upd, cm, rmin), jnp.where(upd, ci, ridx)

    rmin0 = jnp.full((BLK, 1), jnp.inf, jnp.float32)
    ridx0 = jnp.zeros((BLK, 1), jnp.int32)
    rmin, ridx = lax.fori_loop(0, NKC, step, (rmin0, ridx0))
    idx_ref[...] = ridx
    minv_ref[...] = rmin


def _st_body(z_ref, zq_ref, o_ref):
    z = z_ref[...]
    o_ref[...] = z + (zq_ref[...][:, :DIM] - z)


def _gather_body(nc):
    def body(w_hbm, idx_hbm, out_hbm, idx_v, rows_v, sem):
        wid = lax.axis_index("s") * nc + lax.axis_index("c")
        bpw = ROWS // 32
        base = wid * bpw
        pltpu.sync_copy(idx_hbm.at[pl.ds(base, bpw)], idx_v)
        pltpu.async_copy(w_hbm.at[idx_v], rows_v, sem).wait()
        pltpu.sync_copy(rows_v, out_hbm.at[pl.ds(base, bpw)])
    return body


def kernel(z, W):
    B, T, D = z.shape
    flat_z = z.reshape(-1, D)
    zsq = jnp.sum(flat_z ** 2, axis=1, keepdims=True)
    wsq = jnp.sum(W ** 2, axis=1)
    z2b = (2.0 * flat_z).astype(jnp.bfloat16)
    wt3 = (W.T.reshape(DIM, NKC, KC).transpose(1, 0, 2)
           ).astype(jnp.bfloat16)
    wsq3 = wsq.reshape(NKC, 1, KC)

    idx, minv = pl.pallas_call(
        _argmin_body,
        grid=(NBLK,),
        in_specs=[
            pl.BlockSpec((BLK, DIM), lambda i: (i, 0)),
            pl.BlockSpec((BLK, 1), lambda i: (i, 0)),
            pl.BlockSpec((NKC, DIM, KC), lambda i: (0, 0, 0)),
            pl.BlockSpec((NKC, 1, KC), lambda i: (0, 0, 0)),
        ],
        out_specs=[
            pl.BlockSpec((BLK, 1), lambda i: (i, 0)),
            pl.BlockSpec((BLK, 1), lambda i: (i, 0)),
        ],
        out_shape=[
            jax.ShapeDtypeStruct((ROWS, 1), jnp.int32),
            jax.ShapeDtypeStruct((ROWS, 1), jnp.float32),
        ],
    )(z2b, zsq, wt3, wsq3)

    indices = idx.reshape(B, T)

    w_pad = jnp.zeros((CODEBOOK, 128), jnp.float32).at[:, :DIM].set(W)
    info = plsc.get_sparse_core_info()
    nc, ns = info.num_cores, info.num_subcores
    bpw = ROWS // (nc * ns)
    gather = pl.kernel(
        _gather_body(nc),
        out_type=jax.ShapeDtypeStruct((ROWS, 128), jnp.float32),
        mesh=plsc.VectorSubcoreMesh(core_axis_name="c", subcore_axis_name="s"),
        scratch_types=[
            pltpu.VMEM((bpw,), jnp.int32),
            pltpu.VMEM((bpw, 128), jnp.float32),
            pltpu.SemaphoreType.DMA,
        ],
    )
    zq_flat = gather(w_pad, idx.reshape(ROWS))

    zq_st_flat = pl.pallas_call(
        _st_body,
        grid=(8,),
        in_specs=[
            pl.BlockSpec((ROWS // 8, DIM), lambda i: (i, 0)),
            pl.BlockSpec((ROWS // 8, 128), lambda i: (i, 0)),
        ],
        out_specs=pl.BlockSpec((ROWS // 8, DIM), lambda i: (i, 0)),
        out_shape=jax.ShapeDtypeStruct((ROWS, DIM), jnp.float32),
    )(flat_z, zq_flat)

    mean_sq = jnp.sum(minv) * jnp.float32(1.0 / (ROWS * DIM))
    loss = mean_sq + jnp.float32(0.25) * mean_sq
    return (zq_st_flat.reshape(B, T, D), indices, loss)


# speedup vs baseline: 1.0898x; 1.0898x over previous
import functools

import jax
import jax.numpy as jnp
from jax import lax
from jax.experimental import pallas as pl
from jax.experimental.pallas import tpu as pltpu
from jax.experimental.pallas import tpu_sc as plsc

CODEBOOK = 8192
DIM = 32
ROWS = 16384
BLK = 1024
KC = 1024
NBLK = ROWS // BLK
NKC = CODEBOOK // KC


def _argmin_body(z_ref, zsq_ref, wt_ref, wsq_ref, idx_ref, minv_ref):
    z = z_ref[...]
    zsq = zsq_ref[...]

    def step(j, carry):
        rmin, ridx = carry
        wt = wt_ref[j]
        wsq = wsq_ref[j]
        dist = (zsq - jnp.dot(z, wt, preferred_element_type=jnp.float32)) + wsq
        cm = jnp.min(dist, axis=1, keepdims=True)
        io = lax.broadcasted_iota(jnp.int32, (BLK, KC), 1)
        ci = jnp.min(jnp.where(dist == cm, io, CODEBOOK),
                     axis=1, keepdims=True) + j * KC
        upd = cm < rmin
        return jnp.where(upd, cm, rmin), jnp.where(upd, ci, ridx)

    rmin0 = jnp.full((BLK, 1), jnp.inf, jnp.float32)
    ridx0 = jnp.zeros((BLK, 1), jnp.int32)
    rmin, ridx = lax.fori_loop(0, NKC, step, (rmin0, ridx0))
    idx_ref[...] = ridx
    minv_ref[...] = rmin


def _st_body(z_ref, zq_ref, o_ref):
    z = z_ref[...]
    o_ref[...] = z + (zq_ref[...][:, :DIM] - z)


def _gather_body(nc):
    def body(w_hbm, idx_hbm, out_hbm, idx_v, rows_v, sem):
        wid = lax.axis_index("s") * nc + lax.axis_index("c")
        bpw = ROWS // 32
        base = wid * bpw
        pltpu.sync_copy(idx_hbm.at[pl.ds(base, bpw)], idx_v)
        pltpu.async_copy(w_hbm.at[idx_v], rows_v, sem).wait()
        pltpu.sync_copy(rows_v, out_hbm.at[pl.ds(base, bpw)])
    return body


def kernel(z, W):
    B, T, D = z.shape
    flat_z = z.reshape(-1, D)
    zsq = jnp.sum(flat_z ** 2, axis=1, keepdims=True)
    wsq = jnp.sum(W ** 2, axis=1)
    z2b = (2.0 * flat_z).astype(jnp.bfloat16)
    wt3 = (W.T.reshape(DIM, NKC, KC).transpose(1, 0, 2)
           ).astype(jnp.bfloat16)
    wsq3 = wsq.reshape(NKC, 1, KC)

    idx, minv = pl.pallas_call(
        _argmin_body,
        grid=(NBLK,),
        in_specs=[
            pl.BlockSpec((BLK, DIM), lambda i: (i, 0)),
            pl.BlockSpec((BLK, 1), lambda i: (i, 0)),
            pl.BlockSpec((NKC, DIM, KC), lambda i: (0, 0, 0)),
            pl.BlockSpec((NKC, 1, KC), lambda i: (0, 0, 0)),
        ],
        out_specs=[
            pl.BlockSpec((BLK, 1), lambda i: (i, 0)),
            pl.BlockSpec((BLK, 1), lambda i: (i, 0)),
        ],
        out_shape=[
            jax.ShapeDtypeStruct((ROWS, 1), jnp.int32),
            jax.ShapeDtypeStruct((ROWS, 1), jnp.float32),
        ],
    )(z2b, zsq, wt3, wsq3)

    indices = idx.reshape(B, T)

    w_pad = jnp.zeros((CODEBOOK, 128), jnp.float32).at[:, :DIM].set(W)
    info = plsc.get_sparse_core_info()
    nc, ns = info.num_cores, info.num_subcores
    bpw = ROWS // (nc * ns)
    gather = pl.kernel(
        _gather_body(nc),
        out_type=jax.ShapeDtypeStruct((ROWS, 128), jnp.float32),
        mesh=plsc.VectorSubcoreMesh(core_axis_name="c", subcore_axis_name="s"),
        scratch_types=[
            pltpu.VMEM((bpw,), jnp.int32),
            pltpu.VMEM((bpw, 128), jnp.float32),
            pltpu.SemaphoreType.DMA,
        ],
    )
    zq_flat = gather(w_pad, idx.reshape(ROWS))

    zq_st_flat = pl.pallas_call(
        _st_body,
        grid=(8,),
        in_specs=[
            pl.BlockSpec((ROWS // 8, DIM), lambda i: (i, 0)),
            pl.BlockSpec((ROWS // 8, 128), lambda i: (i, 0)),
        ],
        out_specs=pl.BlockSpec((ROWS // 8, DIM), lambda i: (i, 0)),
        out_shape=jax.ShapeDtypeStruct((ROWS, DIM), jnp.float32),
    )(flat_z, zq_flat)

    mean_sq = jnp.sum(minv) * jnp.float32(1.0 / (ROWS * DIM))
    loss = mean_sq + jnp.float32(0.25) * mean_sq
    return (zq_st_flat.reshape(B, T, D), indices, loss)


# speedup vs baseline: 1.1981x; 1.0993x over previous
import functools

import jax
import jax.numpy as jnp
from jax import lax
from jax.experimental import pallas as pl
from jax.experimental.pallas import tpu as pltpu
from jax.experimental.pallas import tpu_sc as plsc

CODEBOOK = 8192
DIM = 32
ROWS = 16384
BLK = 1024
KC = 2048
NBLK = ROWS // BLK
NKC = CODEBOOK // KC


def _argmin_body(z_ref, zsq_ref, wt_ref, wsq_ref, idx_ref, minv_ref):
    z = z_ref[...]
    zsq = zsq_ref[...]

    def step(j, carry):
        rmin, ridx = carry
        wt = wt_ref[j]
        wsq = wsq_ref[j]
        dist = (zsq - jnp.dot(z, wt, preferred_element_type=jnp.float32)) + wsq
        cm = jnp.min(dist, axis=1, keepdims=True)
        io = lax.broadcasted_iota(jnp.int32, (BLK, KC), 1)
        ci = jnp.min(jnp.where(dist == cm, io, CODEBOOK),
                     axis=1, keepdims=True) + j * KC
        upd = cm < rmin
        return jnp.where(upd, cm, rmin), jnp.where(upd, ci, ridx)

    rmin0 = jnp.full((BLK, 1), jnp.inf, jnp.float32)
    ridx0 = jnp.zeros((BLK, 1), jnp.int32)
    rmin, ridx = lax.fori_loop(0, NKC, step, (rmin0, ridx0))
    idx_ref[...] = ridx
    minv_ref[...] = rmin


def _st_body(z_ref, zq_ref, o_ref):
    z = z_ref[...]
    o_ref[...] = z + (zq_ref[...][:, :DIM] - z)


def _gather_body(nc):
    def body(w_hbm, idx_hbm, out_hbm, idx_v, rows_v, sem):
        wid = lax.axis_index("s") * nc + lax.axis_index("c")
        bpw = ROWS // 32
        base = wid * bpw
        pltpu.sync_copy(idx_hbm.at[pl.ds(base, bpw)], idx_v)
        pltpu.async_copy(w_hbm.at[idx_v], rows_v, sem).wait()
        pltpu.sync_copy(rows_v, out_hbm.at[pl.ds(base, bpw)])
    return body


def kernel(z, W):
    B, T, D = z.shape
    flat_z = z.reshape(-1, D)
    zsq = jnp.sum(flat_z ** 2, axis=1, keepdims=True)
    wsq = jnp.sum(W ** 2, axis=1)
    z2b = (2.0 * flat_z).astype(jnp.bfloat16)
    wt3 = (W.T.reshape(DIM, NKC, KC).transpose(1, 0, 2)
           ).astype(jnp.bfloat16)
    wsq3 = wsq.reshape(NKC, 1, KC)

    idx, minv = pl.pallas_call(
        _argmin_body,
        grid=(NBLK,),
        in_specs=[
            pl.BlockSpec((BLK, DIM), lambda i: (i, 0)),
            pl.BlockSpec((BLK, 1), lambda i: (i, 0)),
            pl.BlockSpec((NKC, DIM, KC), lambda i: (0, 0, 0)),
            pl.BlockSpec((NKC, 1, KC), lambda i: (0, 0, 0)),
        ],
        out_specs=[
            pl.BlockSpec((BLK, 1), lambda i: (i, 0)),
            pl.BlockSpec((BLK, 1), lambda i: (i, 0)),
        ],
        out_shape=[
            jax.ShapeDtypeStruct((ROWS, 1), jnp.int32),
            jax.ShapeDtypeStruct((ROWS, 1), jnp.float32),
        ],
    )(z2b, zsq, wt3, wsq3)

    indices = idx.reshape(B, T)

    w_pad = jnp.zeros((CODEBOOK, 128), jnp.float32).at[:, :DIM].set(W)
    info = plsc.get_sparse_core_info()
    nc, ns = info.num_cores, info.num_subcores
    bpw = ROWS // (nc * ns)
    gather = pl.kernel(
        _gather_body(nc),
        out_type=jax.ShapeDtypeStruct((ROWS, 128), jnp.float32),
        mesh=plsc.VectorSubcoreMesh(core_axis_name="c", subcore_axis_name="s"),
        scratch_types=[
            pltpu.VMEM((bpw,), jnp.int32),
            pltpu.VMEM((bpw, 128), jnp.float32),
            pltpu.SemaphoreType.DMA,
        ],
    )
    zq_flat = gather(w_pad, idx.reshape(ROWS))

    zq_st_flat = pl.pallas_call(
        _st_body,
        grid=(8,),
        in_specs=[
            pl.BlockSpec((ROWS // 8, DIM), lambda i: (i, 0)),
            pl.BlockSpec((ROWS // 8, 128), lambda i: (i, 0)),
        ],
        out_specs=pl.BlockSpec((ROWS // 8, DIM), lambda i: (i, 0)),
        out_shape=jax.ShapeDtypeStruct((ROWS, DIM), jnp.float32),
    )(flat_z, zq_flat)

    mean_sq = jnp.sum(minv) * jnp.float32(1.0 / (ROWS * DIM))
    loss = mean_sq + jnp.float32(0.25) * mean_sq
    return (zq_st_flat.reshape(B, T, D), indices, loss)


# speedup vs baseline: 1.2727x; 1.0623x over previous
import functools

import jax
import jax.numpy as jnp
from jax import lax
from jax.experimental import pallas as pl
from jax.experimental.pallas import tpu as pltpu
from jax.experimental.pallas import tpu_sc as plsc

CODEBOOK = 8192
DIM = 32
ROWS = 16384
BLK = 1024
KC = 4096
NBLK = ROWS // BLK
NKC = CODEBOOK // KC


def _argmin_body(z_ref, zsq_ref, wt_ref, wsq_ref, idx_ref, minv_ref):
    z = z_ref[...]
    zsq = zsq_ref[...]

    def step(j, carry):
        rmin, ridx = carry
        wt = wt_ref[j]
        wsq = wsq_ref[j]
        dist = (zsq - jnp.dot(z, wt, preferred_element_type=jnp.float32)) + wsq
        cm = jnp.min(dist, axis=1, keepdims=True)
        io = lax.broadcasted_iota(jnp.int32, (BLK, KC), 1)
        ci = jnp.min(jnp.where(dist == cm, io, CODEBOOK),
                     axis=1, keepdims=True) + j * KC
        upd = cm < rmin
        return jnp.where(upd, cm, rmin), jnp.where(upd, ci, ridx)

    rmin0 = jnp.full((BLK, 1), jnp.inf, jnp.float32)
    ridx0 = jnp.zeros((BLK, 1), jnp.int32)
    rmin, ridx = lax.fori_loop(0, NKC, step, (rmin0, ridx0))
    idx_ref[...] = ridx
    minv_ref[...] = rmin


def _st_body(z_ref, zq_ref, o_ref):
    z = z_ref[...]
    o_ref[...] = z + (zq_ref[...][:, :DIM] - z)


def _gather_body(nc):
    def body(w_hbm, idx_hbm, out_hbm, idx_v, rows_v, sem):
        wid = lax.axis_index("s") * nc + lax.axis_index("c")
        bpw = ROWS // 32
        base = wid * bpw
        pltpu.sync_copy(idx_hbm.at[pl.ds(base, bpw)], idx_v)
        pltpu.async_copy(w_hbm.at[idx_v], rows_v, sem).wait()
        pltpu.sync_copy(rows_v, out_hbm.at[pl.ds(base, bpw)])
    return body


def kernel(z, W):
    B, T, D = z.shape
    flat_z = z.reshape(-1, D)
    zsq = jnp.sum(flat_z ** 2, axis=1, keepdims=True)
    wsq = jnp.sum(W ** 2, axis=1)
    z2b = (2.0 * flat_z).astype(jnp.bfloat16)
    wt3 = (W.T.reshape(DIM, NKC, KC).transpose(1, 0, 2)
           ).astype(jnp.bfloat16)
    wsq3 = wsq.reshape(NKC, 1, KC)

    idx, minv = pl.pallas_call(
        _argmin_body,
        grid=(NBLK,),
        in_specs=[
            pl.BlockSpec((BLK, DIM), lambda i: (i, 0)),
            pl.BlockSpec((BLK, 1), lambda i: (i, 0)),
            pl.BlockSpec((NKC, DIM, KC), lambda i: (0, 0, 0)),
            pl.BlockSpec((NKC, 1, KC), lambda i: (0, 0, 0)),
        ],
        out_specs=[
            pl.BlockSpec((BLK, 1), lambda i: (i, 0)),
            pl.BlockSpec((BLK, 1), lambda i: (i, 0)),
        ],
        out_shape=[
            jax.ShapeDtypeStruct((ROWS, 1), jnp.int32),
            jax.ShapeDtypeStruct((ROWS, 1), jnp.float32),
        ],
    )(z2b, zsq, wt3, wsq3)

    indices = idx.reshape(B, T)

    w_pad = jnp.zeros((CODEBOOK, 128), jnp.float32).at[:, :DIM].set(W)
    info = plsc.get_sparse_core_info()
    nc, ns = info.num_cores, info.num_subcores
    bpw = ROWS // (nc * ns)
    gather = pl.kernel(
        _gather_body(nc),
        out_type=jax.ShapeDtypeStruct((ROWS, 128), jnp.float32),
        mesh=plsc.VectorSubcoreMesh(core_axis_name="c", subcore_axis_name="s"),
        scratch_types=[
            pltpu.VMEM((bpw,), jnp.int32),
            pltpu.VMEM((bpw, 128), jnp.float32),
            pltpu.SemaphoreType.DMA,
        ],
    )
    zq_flat = gather(w_pad, idx.reshape(ROWS))

    zq_st_flat = pl.pallas_call(
        _st_body,
        grid=(8,),
        in_specs=[
            pl.BlockSpec((ROWS // 8, DIM), lambda i: (i, 0)),
            pl.BlockSpec((ROWS // 8, 128), lambda i: (i, 0)),
        ],
        out_specs=pl.BlockSpec((ROWS // 8, DIM), lambda i: (i, 0)),
        out_shape=jax.ShapeDtypeStruct((ROWS, DIM), jnp.float32),
    )(flat_z, zq_flat)

    mean_sq = jnp.sum(minv) * jnp.float32(1.0 / (ROWS * DIM))
    loss = mean_sq + jnp.float32(0.25) * mean_sq
    return (zq_st_flat.reshape(B, T, D), indices, loss)


# speedup vs baseline: 1.3317x; 1.0463x over previous
import functools

import jax
import jax.numpy as jnp
from jax import lax
from jax.experimental import pallas as pl
from jax.experimental.pallas import tpu as pltpu
from jax.experimental.pallas import tpu_sc as plsc

CODEBOOK = 8192
DIM = 32
ROWS = 16384
BLK = 1024
KC = 8192
NBLK = ROWS // BLK
NKC = CODEBOOK // KC


def _argmin_body(z_ref, zsq_ref, wt_ref, wsq_ref, idx_ref, minv_ref):
    z = z_ref[...]
    zsq = zsq_ref[...]

    def step(j, carry):
        rmin, ridx = carry
        wt = wt_ref[j]
        wsq = wsq_ref[j]
        dist = (zsq - jnp.dot(z, wt, preferred_element_type=jnp.float32)) + wsq
        cm = jnp.min(dist, axis=1, keepdims=True)
        io = lax.broadcasted_iota(jnp.int32, (BLK, KC), 1)
        ci = jnp.min(jnp.where(dist == cm, io, CODEBOOK),
                     axis=1, keepdims=True) + j * KC
        upd = cm < rmin
        return jnp.where(upd, cm, rmin), jnp.where(upd, ci, ridx)

    rmin0 = jnp.full((BLK, 1), jnp.inf, jnp.float32)
    ridx0 = jnp.zeros((BLK, 1), jnp.int32)
    rmin, ridx = lax.fori_loop(0, NKC, step, (rmin0, ridx0))
    idx_ref[...] = ridx
    minv_ref[...] = rmin


def _st_body(z_ref, zq_ref, o_ref):
    z = z_ref[...]
    o_ref[...] = z + (zq_ref[...][:, :DIM] - z)


def _gather_body(nc):
    def body(w_hbm, idx_hbm, out_hbm, idx_v, rows_v, sem):
        wid = lax.axis_index("s") * nc + lax.axis_index("c")
        bpw = ROWS // 32
        base = wid * bpw
        pltpu.sync_copy(idx_hbm.at[pl.ds(base, bpw)], idx_v)
        pltpu.async_copy(w_hbm.at[idx_v], rows_v, sem).wait()
        pltpu.sync_copy(rows_v, out_hbm.at[pl.ds(base, bpw)])
    return body


def kernel(z, W):
    B, T, D = z.shape
    flat_z = z.reshape(-1, D)
    zsq = jnp.sum(flat_z ** 2, axis=1, keepdims=True)
    wsq = jnp.sum(W ** 2, axis=1)
    z2b = (2.0 * flat_z).astype(jnp.bfloat16)
    wt3 = (W.T.reshape(DIM, NKC, KC).transpose(1, 0, 2)
           ).astype(jnp.bfloat16)
    wsq3 = wsq.reshape(NKC, 1, KC)

    idx, minv = pl.pallas_call(
        _argmin_body,
        grid=(NBLK,),
        in_specs=[
            pl.BlockSpec((BLK, DIM), lambda i: (i, 0)),
            pl.BlockSpec((BLK, 1), lambda i: (i, 0)),
            pl.BlockSpec((NKC, DIM, KC), lambda i: (0, 0, 0)),
            pl.BlockSpec((NKC, 1, KC), lambda i: (0, 0, 0)),
        ],
        out_specs=[
            pl.BlockSpec((BLK, 1), lambda i: (i, 0)),
            pl.BlockSpec((BLK, 1), lambda i: (i, 0)),
        ],
        out_shape=[
            jax.ShapeDtypeStruct((ROWS, 1), jnp.int32),
            jax.ShapeDtypeStruct((ROWS, 1), jnp.float32),
        ],
    )(z2b, zsq, wt3, wsq3)

    indices = idx.reshape(B, T)

    w_pad = jnp.zeros((CODEBOOK, 128), jnp.float32).at[:, :DIM].set(W)
    info = plsc.get_sparse_core_info()
    nc, ns = info.num_cores, info.num_subcores
    bpw = ROWS // (nc * ns)
    gather = pl.kernel(
        _gather_body(nc),
        out_type=jax.ShapeDtypeStruct((ROWS, 128), jnp.float32),
        mesh=plsc.VectorSubcoreMesh(core_axis_name="c", subcore_axis_name="s"),
        scratch_types=[
            pltpu.VMEM((bpw,), jnp.int32),
            pltpu.VMEM((bpw, 128), jnp.float32),
            pltpu.SemaphoreType.DMA,
        ],
    )
    zq_flat = gather(w_pad, idx.reshape(ROWS))

    zq_st_flat = pl.pallas_call(
        _st_body,
        grid=(8,),
        in_specs=[
            pl.BlockSpec((ROWS // 8, DIM), lambda i: (i, 0)),
            pl.BlockSpec((ROWS // 8, 128), lambda i: (i, 0)),
        ],
        out_specs=pl.BlockSpec((ROWS // 8, DIM), lambda i: (i, 0)),
        out_shape=jax.ShapeDtypeStruct((ROWS, DIM), jnp.float32),
    )(flat_z, zq_flat)

    mean_sq = jnp.sum(minv) * jnp.float32(1.0 / (ROWS * DIM))
    loss = mean_sq + jnp.float32(0.25) * mean_sq
    return (zq_st_flat.reshape(B, T, D), indices, loss)
